# baseline (speedup 1.0000x reference)
import functools

import jax
import jax.numpy as jnp
from jax import lax
from jax.experimental import pallas as pl
from jax.experimental.pallas import tpu as pltpu
from jax.experimental.pallas import tpu_sc as plsc

_N_BINS = 51
_D = 768
_BLK = 1024
_SC_BLOCKS = 6


@functools.partial(jax.jit, static_argnames=("n_tokens",))
def _sc_lookup(flat_values, table, *, n_tokens):
    info = plsc.get_sparse_core_info()
    nc, ns, lanes = info.num_cores, info.num_subcores, info.num_lanes
    nw = nc * ns
    bpw = n_tokens // nw
    chunk = 64
    n_chunks = bpw // chunk

    mesh = plsc.VectorSubcoreMesh(core_axis_name="c", subcore_axis_name="s")

    @functools.partial(
        pl.kernel,
        mesh=mesh,
        out_type=jax.ShapeDtypeStruct((n_tokens, _D), jnp.float32),
        scratch_types=[
            pltpu.VMEM((bpw,), jnp.float32),
            pltpu.VMEM((bpw,), jnp.int32),
            pltpu.VMEM((2, chunk, _D), jnp.float32),
            pltpu.SemaphoreType.DMA,
            pltpu.SemaphoreType.DMA,
        ],
    )
    def sc_kernel(vals_hbm, table_hbm, out_hbm, vals_v, idx_v, rows_v,
                  sem_g, sem_o):
        wid = lax.axis_index("s") * nc + lax.axis_index("c")
        base = wid * bpw
        pltpu.sync_copy(vals_hbm.at[pl.ds(base, bpw)], vals_v)

        def cvt(i, carry):
            v = vals_v[pl.ds(i * lanes, lanes)]
            b = (v * (_N_BINS - 1)).astype(jnp.int32)
            idx_v[pl.ds(i * lanes, lanes)] = jnp.clip(b, 0, _N_BINS - 1)
            return carry

        lax.fori_loop(0, bpw // lanes, cvt, 0, unroll=4)

        def gather(c, buf):
            return pltpu.async_copy(
                table_hbm.at[idx_v.at[pl.ds(c * chunk, chunk)]],
                rows_v.at[buf], sem_g)

        h_g = [gather(0, 0), None]
        h_o = [None, None]
        for c in range(n_chunks):
            buf, nbuf = c & 1, (c + 1) & 1
            if c + 1 < n_chunks:
                if h_o[nbuf] is not None:
                    h_o[nbuf].wait()
                h_g[nbuf] = gather(c + 1, nbuf)
            h_g[buf].wait()
            h_o[buf] = pltpu.async_copy(
                rows_v.at[buf], out_hbm.at[pl.ds(base + c * chunk, chunk)],
                sem_o)
        h_o[0].wait()
        h_o[1].wait()

    return sc_kernel(flat_values, table)


def _tc_body(v_ref, t_ref, o_ref):
    v = v_ref[0, 0, :]
    b = jnp.clip((v * (_N_BINS - 1)).astype(jnp.int32), 0, _N_BINS - 1)
    iota = lax.broadcasted_iota(jnp.int32, (_BLK, 64), 1)
    oh = (b[:, None] == iota).astype(jnp.float32)
    o_ref[0] = jnp.dot(oh, t_ref[...], preferred_element_type=jnp.float32)


def _tc_lookup(vals3d, table_pad):
    n_blocks = vals3d.shape[0]
    return pl.pallas_call(
        _tc_body,
        grid=(n_blocks,),
        in_specs=[
            pl.BlockSpec((1, 1, _BLK), lambda i: (i, 0, 0)),
            pl.BlockSpec((64, _D), lambda i: (0, 0)),
        ],
        out_specs=pl.BlockSpec((1, _BLK, _D), lambda i: (i, 0, 0)),
        out_shape=jax.ShapeDtypeStruct((n_blocks, _BLK, _D), jnp.float32),
    )(vals3d, table_pad)


def kernel(values, embedding_weight):
    batch, seq = values.shape
    n = batch * seq
    n_sc = _SC_BLOCKS * _BLK
    flat = values.reshape(n)

    out_sc = _sc_lookup(flat[n - n_sc:], embedding_weight, n_tokens=n_sc)

    vals3d = flat[: n - n_sc].reshape((n - n_sc) // _BLK, 1, _BLK)
    table_pad = jnp.pad(embedding_weight, ((0, 64 - _N_BINS), (0, 0)))
    out_tc = _tc_lookup(vals3d, table_pad)

    out = jnp.concatenate(
        [out_tc, out_sc.reshape(_SC_BLOCKS, _BLK, _D)], axis=0)
    return out.reshape(batch, seq, _D)


# speedup vs baseline: 1.7440x; 1.7440x over previous
import functools

import jax
import jax.numpy as jnp
from jax import lax
from jax.experimental import pallas as pl
from jax.experimental.pallas import tpu as pltpu
from jax.experimental.pallas import tpu_sc as plsc

_N_BINS = 51
_D = 768
_BLK = 1024
_SC_BLOCKS = 4
_N_BLOCKS = 32


@functools.partial(jax.jit, static_argnames=("n_tokens",))
def _sc_lookup(flat_values, table, *, n_tokens):
    info = plsc.get_sparse_core_info()
    nc, ns, lanes = info.num_cores, info.num_subcores, info.num_lanes
    nw = nc * ns
    bpw = n_tokens // nw
    chunk = 64
    n_chunks = bpw // chunk

    mesh = plsc.VectorSubcoreMesh(core_axis_name="c", subcore_axis_name="s")

    @functools.partial(
        pl.kernel,
        mesh=mesh,
        out_type=jax.ShapeDtypeStruct((n_tokens, _D), jnp.float32),
        scratch_types=[
            pltpu.VMEM((bpw,), jnp.float32),
            pltpu.VMEM((bpw,), jnp.int32),
            pltpu.VMEM((2, chunk, _D), jnp.float32),
            pltpu.SemaphoreType.DMA,
            pltpu.SemaphoreType.DMA,
        ],
    )
    def sc_kernel(vals_hbm, table_hbm, out_hbm, vals_v, idx_v, rows_v,
                  sem_g, sem_o):
        wid = lax.axis_index("s") * nc + lax.axis_index("c")
        base = wid * bpw
        pltpu.sync_copy(vals_hbm.at[pl.ds(base, bpw)], vals_v)

        def cvt(i, carry):
            v = vals_v[pl.ds(i * lanes, lanes)]
            b = (v * (_N_BINS - 1)).astype(jnp.int32)
            idx_v[pl.ds(i * lanes, lanes)] = jnp.clip(b, 0, _N_BINS - 1)
            return carry

        lax.fori_loop(0, bpw // lanes, cvt, 0, unroll=4)

        def gather(c, buf):
            return pltpu.async_copy(
                table_hbm.at[idx_v.at[pl.ds(c * chunk, chunk)]],
                rows_v.at[buf], sem_g)

        h_g = [gather(0, 0), None]
        h_o = [None, None]
        for c in range(n_chunks):
            buf, nbuf = c & 1, (c + 1) & 1
            if c + 1 < n_chunks:
                if h_o[nbuf] is not None:
                    h_o[nbuf].wait()
                h_g[nbuf] = gather(c + 1, nbuf)
            h_g[buf].wait()
            h_o[buf] = pltpu.async_copy(
                rows_v.at[buf], out_hbm.at[pl.ds(base + c * chunk, chunk)],
                sem_o)
        h_o[0].wait()
        h_o[1].wait()

    return sc_kernel(flat_values, table)


def _tc_body(v_ref, t_ref, o_ref):
    v = v_ref[0, 0, :]
    b = jnp.clip((v * (_N_BINS - 1)).astype(jnp.int32), 0, _N_BINS - 1)
    iota = lax.broadcasted_iota(jnp.int32, (_BLK, 64), 1)
    oh = (b[:, None] == iota).astype(jnp.float32)
    o_ref[0] = jnp.dot(oh, t_ref[...], preferred_element_type=jnp.float32)


def _tc_lookup(vals3d, table_pad):
    n_blocks = vals3d.shape[0]
    return pl.pallas_call(
        _tc_body,
        grid=(n_blocks,),
        in_specs=[
            pl.BlockSpec((1, 1, _BLK), lambda i: (i, 0, 0)),
            pl.BlockSpec((64, _D), lambda i: (0, 0)),
        ],
        out_specs=pl.BlockSpec((1, _BLK, _D), lambda i: (i, 0, 0)),
        out_shape=jax.ShapeDtypeStruct((_N_BLOCKS, _BLK, _D), jnp.float32),
    )(vals3d, table_pad)


def _merge_body(big_ref, sc_ref, o_ref):
    del big_ref
    o_ref[0] = sc_ref[0]


def _merge(big, sc3d):
    n_tc = _N_BLOCKS - _SC_BLOCKS
    return pl.pallas_call(
        _merge_body,
        grid=(_SC_BLOCKS,),
        in_specs=[
            pl.BlockSpec(memory_space=pl.ANY),
            pl.BlockSpec((1, _BLK, _D), lambda i: (i, 0, 0)),
        ],
        out_specs=pl.BlockSpec((1, _BLK, _D), lambda i: (n_tc + i, 0, 0)),
        out_shape=jax.ShapeDtypeStruct((_N_BLOCKS, _BLK, _D), jnp.float32),
        input_output_aliases={0: 0},
    )(big, sc3d)


def kernel(values, embedding_weight):
    batch, seq = values.shape
    n = batch * seq
    n_sc = _SC_BLOCKS * _BLK
    flat = values.reshape(n)

    out_sc = _sc_lookup(flat[n - n_sc:], embedding_weight, n_tokens=n_sc)

    vals3d = flat[: n - n_sc].reshape((n - n_sc) // _BLK, 1, _BLK)
    table_pad = jnp.pad(embedding_weight, ((0, 64 - _N_BINS), (0, 0)))
    out_tc = _tc_lookup(vals3d, table_pad)

    out = _merge(out_tc, out_sc.reshape(_SC_BLOCKS, _BLK, _D))
    return out.reshape(batch, seq, _D)


# speedup vs baseline: 2.0345x; 1.1666x over previous
import functools

import jax
import jax.numpy as jnp
from jax import lax
from jax.experimental import pallas as pl
from jax.experimental.pallas import tpu as pltpu
from jax.experimental.pallas import tpu_sc as plsc

_N_BINS = 51
_D = 768
_BLK = 1024
_SC_BLOCKS = 2
_N_BLOCKS = 32


@functools.partial(jax.jit, static_argnames=("n_tokens",))
def _sc_lookup(flat_values, table, *, n_tokens):
    info = plsc.get_sparse_core_info()
    nc, ns, lanes = info.num_cores, info.num_subcores, info.num_lanes
    nw = nc * ns
    bpw = n_tokens // nw
    chunk = min(64, bpw)
    n_chunks = bpw // chunk

    mesh = plsc.VectorSubcoreMesh(core_axis_name="c", subcore_axis_name="s")

    @functools.partial(
        pl.kernel,
        mesh=mesh,
        out_type=jax.ShapeDtypeStruct((n_tokens, _D), jnp.float32),
        scratch_types=[
            pltpu.VMEM((bpw,), jnp.float32),
            pltpu.VMEM((bpw,), jnp.int32),
            pltpu.VMEM((2, chunk, _D), jnp.float32),
            pltpu.SemaphoreType.DMA,
            pltpu.SemaphoreType.DMA,
        ],
    )
    def sc_kernel(vals_hbm, table_hbm, out_hbm, vals_v, idx_v, rows_v,
                  sem_g, sem_o):
        wid = lax.axis_index("s") * nc + lax.axis_index("c")
        base = wid * bpw
        pltpu.sync_copy(vals_hbm.at[pl.ds(base, bpw)], vals_v)

        def cvt(i, carry):
            v = vals_v[pl.ds(i * lanes, lanes)]
            b = (v * (_N_BINS - 1)).astype(jnp.int32)
            idx_v[pl.ds(i * lanes, lanes)] = jnp.clip(b, 0, _N_BINS - 1)
            return carry

        lax.fori_loop(0, bpw // lanes, cvt, 0, unroll=4)

        def gather(c, buf):
            return pltpu.async_copy(
                table_hbm.at[idx_v.at[pl.ds(c * chunk, chunk)]],
                rows_v.at[buf], sem_g)

        h_g = [gather(0, 0), None]
        h_o = [None, None]
        for c in range(n_chunks):
            buf, nbuf = c & 1, (c + 1) & 1
            if c + 1 < n_chunks:
                if h_o[nbuf] is not None:
                    h_o[nbuf].wait()
                h_g[nbuf] = gather(c + 1, nbuf)
            h_g[buf].wait()
            h_o[buf] = pltpu.async_copy(
                rows_v.at[buf], out_hbm.at[pl.ds(base + c * chunk, chunk)],
                sem_o)
        for h in h_o:
            if h is not None:
                h.wait()

    return sc_kernel(flat_values, table)


def _tc_body(v_ref, t_ref, o_ref):
    v = v_ref[0, 0, :]
    b = jnp.clip((v * (_N_BINS - 1)).astype(jnp.int32), 0, _N_BINS - 1)
    iota = lax.broadcasted_iota(jnp.int32, (_BLK, 64), 1)
    oh = (b[:, None] == iota).astype(jnp.float32)
    o_ref[0] = jnp.dot(oh, t_ref[...], preferred_element_type=jnp.float32)


def _tc_lookup(vals3d, table_pad):
    n_blocks = vals3d.shape[0]
    return pl.pallas_call(
        _tc_body,
        grid=(n_blocks,),
        in_specs=[
            pl.BlockSpec((1, 1, _BLK), lambda i: (i, 0, 0)),
            pl.BlockSpec((64, _D), lambda i: (0, 0)),
        ],
        out_specs=pl.BlockSpec((1, _BLK, _D), lambda i: (i, 0, 0)),
        out_shape=jax.ShapeDtypeStruct((_N_BLOCKS, _BLK, _D), jnp.float32),
    )(vals3d, table_pad)


def _merge_body(big_ref, sc_ref, o_ref):
    del big_ref
    o_ref[0] = sc_ref[0]


def _merge(big, sc3d):
    n_tc = _N_BLOCKS - _SC_BLOCKS
    return pl.pallas_call(
        _merge_body,
        grid=(_SC_BLOCKS,),
        in_specs=[
            pl.BlockSpec(memory_space=pl.ANY),
            pl.BlockSpec((1, _BLK, _D), lambda i: (i, 0, 0)),
        ],
        out_specs=pl.BlockSpec((1, _BLK, _D), lambda i: (n_tc + i, 0, 0)),
        out_shape=jax.ShapeDtypeStruct((_N_BLOCKS, _BLK, _D), jnp.float32),
        input_output_aliases={0: 0},
    )(big, sc3d)


def kernel(values, embedding_weight):
    batch, seq = values.shape
    n = batch * seq
    n_sc = _SC_BLOCKS * _BLK
    flat = values.reshape(n)

    out_sc = _sc_lookup(flat[n - n_sc:], embedding_weight, n_tokens=n_sc)

    vals3d = flat[: n - n_sc].reshape((n - n_sc) // _BLK, 1, _BLK)
    table_pad = jnp.pad(embedding_weight, ((0, 64 - _N_BINS), (0, 0)))
    out_tc = _tc_lookup(vals3d, table_pad)

    out = _merge(out_tc, out_sc.reshape(_SC_BLOCKS, _BLK, _D))
    return out.reshape(batch, seq, _D)


# speedup vs baseline: 2.1660x; 1.0646x over previous
import functools

import jax
import jax.numpy as jnp
from jax import lax
from jax.experimental import pallas as pl
from jax.experimental.pallas import tpu as pltpu
from jax.experimental.pallas import tpu_sc as plsc

_N_BINS = 51
_D = 768
_BLK = 1024
_SC_BLOCKS = 1
_N_BLOCKS = 32


@functools.partial(jax.jit, static_argnames=("n_tokens",))
def _sc_lookup(flat_values, table, *, n_tokens):
    info = plsc.get_sparse_core_info()
    nc, ns, lanes = info.num_cores, info.num_subcores, info.num_lanes
    nw = nc * ns
    bpw = n_tokens // nw
    chunk = min(64, bpw)
    n_chunks = bpw // chunk

    mesh = plsc.VectorSubcoreMesh(core_axis_name="c", subcore_axis_name="s")

    @functools.partial(
        pl.kernel,
        mesh=mesh,
        out_type=jax.ShapeDtypeStruct((n_tokens, _D), jnp.float32),
        scratch_types=[
            pltpu.VMEM((bpw,), jnp.float32),
            pltpu.VMEM((bpw,), jnp.int32),
            pltpu.VMEM((2, chunk, _D), jnp.float32),
            pltpu.SemaphoreType.DMA,
            pltpu.SemaphoreType.DMA,
        ],
    )
    def sc_kernel(vals_hbm, table_hbm, out_hbm, vals_v, idx_v, rows_v,
                  sem_g, sem_o):
        wid = lax.axis_index("s") * nc + lax.axis_index("c")
        base = wid * bpw
        pltpu.sync_copy(vals_hbm.at[pl.ds(base, bpw)], vals_v)

        def cvt(i, carry):
            v = vals_v[pl.ds(i * lanes, lanes)]
            b = (v * (_N_BINS - 1)).astype(jnp.int32)
            idx_v[pl.ds(i * lanes, lanes)] = jnp.clip(b, 0, _N_BINS - 1)
            return carry

        lax.fori_loop(0, bpw // lanes, cvt, 0, unroll=4)

        def gather(c, buf):
            return pltpu.async_copy(
                table_hbm.at[idx_v.at[pl.ds(c * chunk, chunk)]],
                rows_v.at[buf], sem_g)

        h_g = [gather(0, 0), None]
        h_o = [None, None]
        for c in range(n_chunks):
            buf, nbuf = c & 1, (c + 1) & 1
            if c + 1 < n_chunks:
                if h_o[nbuf] is not None:
                    h_o[nbuf].wait()
                h_g[nbuf] = gather(c + 1, nbuf)
            h_g[buf].wait()
            h_o[buf] = pltpu.async_copy(
                rows_v.at[buf], out_hbm.at[pl.ds(base + c * chunk, chunk)],
                sem_o)
        for h in h_o:
            if h is not None:
                h.wait()

    return sc_kernel(flat_values, table)


def _tc_body(v_ref, t_ref, o_ref):
    v = v_ref[0, 0, :]
    b = jnp.clip((v * (_N_BINS - 1)).astype(jnp.int32), 0, _N_BINS - 1)
    iota = lax.broadcasted_iota(jnp.int32, (_BLK, 64), 1)
    oh = (b[:, None] == iota).astype(jnp.float32)
    o_ref[0] = jnp.dot(oh, t_ref[...], preferred_element_type=jnp.float32)


def _tc_lookup(vals3d, table_pad):
    n_blocks = vals3d.shape[0]
    return pl.pallas_call(
        _tc_body,
        grid=(n_blocks,),
        in_specs=[
            pl.BlockSpec((1, 1, _BLK), lambda i: (i, 0, 0)),
            pl.BlockSpec((64, _D), lambda i: (0, 0)),
        ],
        out_specs=pl.BlockSpec((1, _BLK, _D), lambda i: (i, 0, 0)),
        out_shape=jax.ShapeDtypeStruct((_N_BLOCKS, _BLK, _D), jnp.float32),
    )(vals3d, table_pad)


def _merge_body(big_ref, sc_ref, o_ref):
    del big_ref
    o_ref[0] = sc_ref[0]


def _merge(big, sc3d):
    n_tc = _N_BLOCKS - _SC_BLOCKS
    return pl.pallas_call(
        _merge_body,
        grid=(_SC_BLOCKS,),
        in_specs=[
            pl.BlockSpec(memory_space=pl.ANY),
            pl.BlockSpec((1, _BLK, _D), lambda i: (i, 0, 0)),
        ],
        out_specs=pl.BlockSpec((1, _BLK, _D), lambda i: (n_tc + i, 0, 0)),
        out_shape=jax.ShapeDtypeStruct((_N_BLOCKS, _BLK, _D), jnp.float32),
        input_output_aliases={0: 0},
    )(big, sc3d)


def kernel(values, embedding_weight):
    batch, seq = values.shape
    n = batch * seq
    n_sc = _SC_BLOCKS * _BLK
    flat = values.reshape(n)

    out_sc = _sc_lookup(flat[n - n_sc:], embedding_weight, n_tokens=n_sc)

    vals3d = flat[: n - n_sc].reshape((n - n_sc) // _BLK, 1, _BLK)
    table_pad = jnp.pad(embedding_weight, ((0, 64 - _N_BINS), (0, 0)))
    out_tc = _tc_lookup(vals3d, table_pad)

    out = _merge(out_tc, out_sc.reshape(_SC_BLOCKS, _BLK, _D))
    return out.reshape(batch, seq, _D)


# speedup vs baseline: 2.2319x; 1.0304x over previous
import functools

import jax
import jax.numpy as jnp
from jax import lax
from jax.experimental import pallas as pl
from jax.experimental.pallas import tpu as pltpu
from jax.experimental.pallas import tpu_sc as plsc

_N_BINS = 51
_D = 768
_BLK = 1024
_SC_BLOCKS = 1
_N_BLOCKS = 32


@functools.partial(jax.jit, static_argnames=("n_tokens",))
def _sc_lookup(flat_values, table, *, n_tokens):
    info = plsc.get_sparse_core_info()
    nc, ns, lanes = 1, info.num_subcores, info.num_lanes
    nw = nc * ns
    bpw = n_tokens // nw
    chunk = min(64, bpw)
    n_chunks = bpw // chunk

    mesh = plsc.VectorSubcoreMesh(
        core_axis_name="c", subcore_axis_name="s", num_cores=1)

    @functools.partial(
        pl.kernel,
        mesh=mesh,
        out_type=jax.ShapeDtypeStruct((n_tokens, _D), jnp.float32),
        scratch_types=[
            pltpu.VMEM((bpw,), jnp.float32),
            pltpu.VMEM((bpw,), jnp.int32),
            pltpu.VMEM((2, chunk, _D), jnp.float32),
            pltpu.SemaphoreType.DMA,
            pltpu.SemaphoreType.DMA,
        ],
    )
    def sc_kernel(vals_hbm, table_hbm, out_hbm, vals_v, idx_v, rows_v,
                  sem_g, sem_o):
        wid = lax.axis_index("s") * nc + lax.axis_index("c")
        base = wid * bpw
        pltpu.sync_copy(vals_hbm.at[pl.ds(base, bpw)], vals_v)

        def cvt(i, carry):
            v = vals_v[pl.ds(i * lanes, lanes)]
            b = (v * (_N_BINS - 1)).astype(jnp.int32)
            idx_v[pl.ds(i * lanes, lanes)] = jnp.clip(b, 0, _N_BINS - 1)
            return carry

        lax.fori_loop(0, bpw // lanes, cvt, 0, unroll=4)

        def gather(c, buf):
            return pltpu.async_copy(
                table_hbm.at[idx_v.at[pl.ds(c * chunk, chunk)]],
                rows_v.at[buf], sem_g)

        h_g = [gather(0, 0), None]
        h_o = [None, None]
        for c in range(n_chunks):
            buf, nbuf = c & 1, (c + 1) & 1
            if c + 1 < n_chunks:
                if h_o[nbuf] is not None:
                    h_o[nbuf].wait()
                h_g[nbuf] = gather(c + 1, nbuf)
            h_g[buf].wait()
            h_o[buf] = pltpu.async_copy(
                rows_v.at[buf], out_hbm.at[pl.ds(base + c * chunk, chunk)],
                sem_o)
        for h in h_o:
            if h is not None:
                h.wait()

    return sc_kernel(flat_values, table)


def _tc_body(v_ref, t_ref, o_ref):
    v = v_ref[0, 0, :]
    b = jnp.clip((v * (_N_BINS - 1)).astype(jnp.int32), 0, _N_BINS - 1)
    iota = lax.broadcasted_iota(jnp.int32, (_BLK, _N_BINS), 1)
    oh = (b[:, None] == iota).astype(jnp.float32)
    o_ref[0] = jnp.dot(oh, t_ref[...], preferred_element_type=jnp.float32)


def _tc_lookup(vals3d, table_pad):
    n_blocks = vals3d.shape[0]
    return pl.pallas_call(
        _tc_body,
        grid=(n_blocks,),
        in_specs=[
            pl.BlockSpec((1, 1, _BLK), lambda i: (i, 0, 0)),
            pl.BlockSpec((_N_BINS, _D), lambda i: (0, 0)),
        ],
        out_specs=pl.BlockSpec((1, _BLK, _D), lambda i: (i, 0, 0)),
        out_shape=jax.ShapeDtypeStruct((_N_BLOCKS, _BLK, _D), jnp.float32),
    )(vals3d, table_pad)


def _merge_body(big_ref, sc_ref, o_ref):
    del big_ref
    o_ref[0] = sc_ref[0]


def _merge(big, sc3d):
    n_tc = _N_BLOCKS - _SC_BLOCKS
    return pl.pallas_call(
        _merge_body,
        grid=(_SC_BLOCKS,),
        in_specs=[
            pl.BlockSpec(memory_space=pl.ANY),
            pl.BlockSpec((1, _BLK, _D), lambda i: (i, 0, 0)),
        ],
        out_specs=pl.BlockSpec((1, _BLK, _D), lambda i: (n_tc + i, 0, 0)),
        out_shape=jax.ShapeDtypeStruct((_N_BLOCKS, _BLK, _D), jnp.float32),
        input_output_aliases={0: 0},
    )(big, sc3d)


def kernel(values, embedding_weight):
    batch, seq = values.shape
    n = batch * seq
    n_sc = _SC_BLOCKS * _BLK
    flat = values.reshape(n)

    out_sc = _sc_lookup(flat[n - n_sc:], embedding_weight, n_tokens=n_sc)

    vals3d = flat[: n - n_sc].reshape((n - n_sc) // _BLK, 1, _BLK)
    out_tc = _tc_lookup(vals3d, embedding_weight)

    out = _merge(out_tc, out_sc.reshape(_SC_BLOCKS, _BLK, _D))
    return out.reshape(batch, seq, _D)
